# baseline (speedup 1.0000x reference)
import functools

import jax
import jax.numpy as jnp
from jax import lax
from jax.experimental import pallas as pl
from jax.experimental.pallas import tpu as pltpu
from jax.experimental.pallas import tpu_sc as plsc

N, T, S, D = 8, 64, 196, 384
SLABS = N * S
NUM_WORKERS = 32
K_SC = 640
PER_W = K_SC // NUM_WORKERS
LANES = 16
DV = D // LANES
RING = 4
ROWS_PER_STEP = 4
TC_BLK = 8


def _sc_body(xt_hbm, temb_hbm, y_hbm, tbuf, b0, b1, b2, b3,
             in0, in1, in2, in3, out0, out1, out2, out3):
    cid = lax.axis_index("c")
    sid = lax.axis_index("s")
    w = sid * 2 + cid

    pltpu.sync_copy(temb_hbm, tbuf)

    bufs = (b0, b1, b2, b3)
    in_sems = (in0, in1, in2, in3)
    out_sems = (out0, out1, out2, out3)

    def start_in(p, idx):
        pltpu.async_copy(xt_hbm.at[w + idx * NUM_WORKERS], bufs[p], in_sems[p])

    def wait_in(p):
        pltpu.make_async_copy(xt_hbm.at[0], bufs[p], in_sems[p]).wait()

    def start_out(p, idx):
        pltpu.async_copy(bufs[p], y_hbm.at[w + idx * NUM_WORKERS], out_sems[p])

    def wait_out(p):
        pltpu.make_async_copy(bufs[p], y_hbm.at[0], out_sems[p]).wait()

    start_in(0, 0)
    start_in(1, 1)

    def stage(p, q, idx):
        wait_in(p)

        @pl.when(idx + 2 < PER_W)
        def _():
            @pl.when(idx >= 2)
            def _():
                wait_out(q)

            start_in(q, idx + 2)

        xb = bufs[p]

        def per_rows(r0, c2):
            row0 = r0 * ROWS_PER_STEP
            for rr in range(ROWS_PER_STEP):
                row = row0 + rr
                for c in range(DV):
                    sl = pl.ds(c * LANES, LANES)
                    plsc.addupdate(xb.at[row, sl], tbuf[row, sl])
            return c2

        lax.fori_loop(0, T // ROWS_PER_STEP, per_rows, 0, unroll=False)
        start_out(p, idx)

    def body(k, carry):
        u0 = k * RING
        for p in range(RING):
            @pl.when(u0 + p < PER_W)
            def _(p=p):
                stage(p, (p + 2) % RING, u0 + p)
        return carry

    lax.fori_loop(0, (PER_W + RING - 1) // RING, body, 0, unroll=False)
    for p in range(RING):
        wait_out(p)


def _sc_add(xt, T_embed):
    mesh = plsc.VectorSubcoreMesh(core_axis_name="c", subcore_axis_name="s")
    fn = pl.kernel(
        _sc_body,
        mesh=mesh,
        compiler_params=pltpu.CompilerParams(use_tc_tiling_on_sc=True),
        out_type=jax.ShapeDtypeStruct((K_SC, T, D), jnp.float32),
        scratch_types=(
            [pltpu.VMEM((T, D), jnp.float32) for _ in range(5)]
            + [pltpu.SemaphoreType.DMA for _ in range(8)]
        ),
    )
    return fn(xt, T_embed)


def _tc_body(x_ref, t_ref, o_ref):
    o_ref[...] = x_ref[...] + t_ref[...][None, :, :]


def _tc_add(xt, T_embed):
    grid = ((SLABS - K_SC) // TC_BLK,)
    return pl.pallas_call(
        _tc_body,
        grid=grid,
        in_specs=[
            pl.BlockSpec((TC_BLK, T, D),
                         lambda i: (i + K_SC // TC_BLK, 0, 0)),
            pl.BlockSpec((T, D), lambda i: (0, 0)),
        ],
        out_specs=pl.BlockSpec((TC_BLK, T, D),
                               lambda i: (i + K_SC // TC_BLK, 0, 0)),
        out_shape=jax.ShapeDtypeStruct((SLABS, T, D), jnp.float32),
    )(xt, T_embed)


@jax.jit
def _hybrid(xt, T_embed):
    y_sc = _sc_add(xt, T_embed)
    full = _tc_add(xt, T_embed)
    return lax.dynamic_update_slice(full, y_sc, (0, 0, 0))


def kernel(x, T_embed):
    n, t_len, s, d = x.shape
    xt = jnp.transpose(x, (0, 2, 1, 3)).reshape(n * s, t_len, d)
    yt = _hybrid(xt, T_embed)
    return jnp.transpose(yt.reshape(n, s, t_len, d), (0, 2, 1, 3))


# speedup vs baseline: 1.3661x; 1.3661x over previous
import jax
import jax.numpy as jnp
from jax import lax
from jax.experimental import pallas as pl
from jax.experimental.pallas import tpu as pltpu
from jax.experimental.pallas import tpu_sc as plsc

N, T, S, D = 8, 64, 196, 384
SLABS = N * S
NUM_WORKERS = 32
PER_W = SLABS // NUM_WORKERS
LANES = 16
DV = D // LANES
RING = 4
ROWS_PER_STEP = 4


def _sc_body(xt_hbm, temb_hbm, y_hbm, tbuf, b0, b1, b2, b3,
             in0, in1, in2, in3, out0, out1, out2, out3):
    cid = lax.axis_index("c")
    sid = lax.axis_index("s")
    w = sid * 2 + cid

    pltpu.sync_copy(temb_hbm, tbuf)

    bufs = (b0, b1, b2, b3)
    in_sems = (in0, in1, in2, in3)
    out_sems = (out0, out1, out2, out3)

    def start_in(p, idx):
        pltpu.async_copy(xt_hbm.at[w + idx * NUM_WORKERS], bufs[p], in_sems[p])

    def wait_in(p):
        pltpu.make_async_copy(xt_hbm.at[0], bufs[p], in_sems[p]).wait()

    def start_out(p, idx):
        pltpu.async_copy(bufs[p], y_hbm.at[w + idx * NUM_WORKERS], out_sems[p])

    def wait_out(p):
        pltpu.make_async_copy(bufs[p], y_hbm.at[0], out_sems[p]).wait()

    start_in(0, 0)
    start_in(1, 1)

    def compute(p):
        xb = bufs[p]

        def per_rows(r0, c2):
            row0 = r0 * ROWS_PER_STEP
            for rr in range(ROWS_PER_STEP):
                row = row0 + rr
                for c in range(DV):
                    sl = pl.ds(c * LANES, LANES)
                    plsc.addupdate(xb.at[row, sl], tbuf[row, sl])
            return c2

        lax.fori_loop(0, T // ROWS_PER_STEP, per_rows, 0, unroll=False)

    def stage(p, idx, q=None, fire=None, wait_prev=False):
        wait_in(p)
        if wait_prev:
            wait_out(q)
        if fire is not None:
            start_in(q, fire)
        compute(p)
        start_out(p, idx)

    stage(0, 0, q=2, fire=2)
    stage(1, 1, q=3, fire=3)

    def body(k, carry):
        idx0 = 2 + k * RING
        for j in range(RING):
            stage((2 + j) % RING, idx0 + j, q=j % RING,
                  fire=idx0 + j + 2, wait_prev=True)
        return carry

    lax.fori_loop(0, (PER_W - 5) // RING, body, 0, unroll=False)

    stage(2, PER_W - 3, q=0, fire=PER_W - 1, wait_prev=True)
    stage(3, PER_W - 2)
    stage(0, PER_W - 1)
    for p in range(RING):
        wait_out(p)


@jax.jit
def _sc_add(xt, T_embed):
    mesh = plsc.VectorSubcoreMesh(core_axis_name="c", subcore_axis_name="s")
    fn = pl.kernel(
        _sc_body,
        mesh=mesh,
        compiler_params=pltpu.CompilerParams(use_tc_tiling_on_sc=True),
        out_type=jax.ShapeDtypeStruct((SLABS, T, D), jnp.float32),
        scratch_types=(
            [pltpu.VMEM((T, D), jnp.float32) for _ in range(5)]
            + [pltpu.SemaphoreType.DMA for _ in range(8)]
        ),
    )
    return fn(xt, T_embed)


def kernel(x, T_embed):
    n, t_len, s, d = x.shape
    xt = jnp.transpose(x, (0, 2, 1, 3)).reshape(n * s, t_len, d)
    yt = _sc_add(xt, T_embed)
    return jnp.transpose(yt.reshape(n, s, t_len, d), (0, 2, 1, 3))


# speedup vs baseline: 1.4168x; 1.0371x over previous
import jax
import jax.numpy as jnp
from jax import lax
from jax.experimental import pallas as pl
from jax.experimental.pallas import tpu as pltpu
from jax.experimental.pallas import tpu_sc as plsc

N, T, S, D = 8, 64, 196, 384
SLABS = N * S
NUM_WORKERS = 32
PER_W = SLABS // NUM_WORKERS
LANES = 16
DV = D // LANES
RING = 4
ROWS_PER_STEP = 4


def _sc_body(xt_hbm, temb_hbm, y_hbm, tbuf, b0, b1, b2, b3,
             in0, in1, in2, in3, out0, out1, out2, out3):
    cid = lax.axis_index("c")
    sid = lax.axis_index("s")
    w = sid * 2 + cid

    pltpu.sync_copy(temb_hbm, tbuf)

    bufs = (b0, b1, b2, b3)
    in_sems = (in0, in1, in2, in3)
    out_sems = (out0, out1, out2, out3)

    def start_in(p, idx):
        pltpu.async_copy(xt_hbm.at[w + idx * NUM_WORKERS], bufs[p], in_sems[p])

    def wait_in(p):
        pltpu.make_async_copy(xt_hbm.at[0], bufs[p], in_sems[p]).wait()

    def start_out(p, idx):
        pltpu.async_copy(bufs[p], y_hbm.at[w + idx * NUM_WORKERS], out_sems[p])

    def wait_out(p):
        pltpu.make_async_copy(bufs[p], y_hbm.at[0], out_sems[p]).wait()

    start_in(0, 0)
    start_in(1, 1)

    def stage(p, q, idx):
        wait_in(p)

        @pl.when(idx + 2 < PER_W)
        def _():
            @pl.when(idx >= 2)
            def _():
                wait_out(q)

            start_in(q, idx + 2)

        xb = bufs[p]

        def per_rows(r0, c2):
            row0 = r0 * ROWS_PER_STEP
            for rr in range(ROWS_PER_STEP):
                row = row0 + rr
                for c in range(DV):
                    sl = pl.ds(c * LANES, LANES)
                    plsc.addupdate(xb.at[row, sl], tbuf[row, sl])
            return c2

        lax.fori_loop(0, T // ROWS_PER_STEP, per_rows, 0, unroll=False)
        start_out(p, idx)

    def body(k, carry):
        u0 = k * RING
        for p in range(RING):
            @pl.when(u0 + p < PER_W)
            def _(p=p):
                stage(p, (p + 2) % RING, u0 + p)
        return carry

    lax.fori_loop(0, (PER_W + RING - 1) // RING, body, 0, unroll=False)
    for p in range(RING):
        wait_out(p)


@jax.jit
def _sc_add(xt, T_embed):
    mesh = plsc.VectorSubcoreMesh(core_axis_name="c", subcore_axis_name="s")
    fn = pl.kernel(
        _sc_body,
        mesh=mesh,
        compiler_params=pltpu.CompilerParams(use_tc_tiling_on_sc=True),
        out_type=jax.ShapeDtypeStruct((SLABS, T, D), jnp.float32),
        scratch_types=(
            [pltpu.VMEM((T, D), jnp.float32) for _ in range(5)]
            + [pltpu.SemaphoreType.DMA for _ in range(8)]
        ),
    )
    return fn(xt, T_embed)


def kernel(x, T_embed):
    n, t_len, s, d = x.shape
    xt = jnp.transpose(x, (0, 2, 1, 3)).reshape(n * s, t_len, d)
    yt = _sc_add(xt, T_embed)
    return jnp.transpose(yt.reshape(n, s, t_len, d), (0, 2, 1, 3))
